# baseline (speedup 1.0000x reference)
import jax
import jax.numpy as jnp
from jax.experimental import pallas as pl

NP_REAL = 8732
NP_PAD = 8832
NUM = 32
NT = 64
V0 = 0.1
V1 = 0.2
F32 = jnp.float32
I32 = jnp.int32


def _mb_kernel(pr_ref, tr_ref, loc_ref, conf_ref, f1_ref, f2_ref, lab_ref,
               out_ref):
    lane = jax.lax.broadcasted_iota(I32, (1, NP_PAD), 1)
    valid = lane < NP_REAL

    cx = pr_ref[0:1, :]
    cy = pr_ref[1:2, :]
    w = pr_ref[2:3, :]
    h = pr_ref[3:4, :]
    px1 = cx - w * 0.5
    py1 = cy - h * 0.5
    px2 = cx + w * 0.5
    py2 = cy + h * 0.5
    area_b = (px2 - px1) * (py2 - py1)

    big = jnp.int32(NP_PAD)

    def body(t, carry):
        bov, bx1, by1, bx2, by2 = carry
        tb = tr_ref[t]
        ax1 = tb[:, 0:1]
        ay1 = tb[:, 1:2]
        ax2 = tb[:, 2:3]
        ay2 = tb[:, 3:4]
        iw = jnp.maximum(jnp.minimum(ax2, px2) - jnp.maximum(ax1, px1), 0.0)
        ih = jnp.maximum(jnp.minimum(ay2, py2) - jnp.maximum(ay1, py1), 0.0)
        inter = iw * ih
        area_a = (ax2 - ax1) * (ay2 - ay1)
        ov = inter / (area_a + area_b - inter)
        m = jnp.max(ov, axis=1, keepdims=True)
        fm = ov == m
        ovf = jnp.where(fm, 2.0 + t.astype(F32) * 0.0009765625, ov)
        upd = ovf > bov
        bov = jnp.where(upd, ovf, bov)
        bx1 = jnp.where(upd, ax1, bx1)
        by1 = jnp.where(upd, ay1, by1)
        bx2 = jnp.where(upd, ax2, bx2)
        by2 = jnp.where(upd, ay2, by2)
        return bov, bx1, by1, bx2, by2

    zeros = jnp.zeros((NUM, NP_PAD), F32)
    init = (jnp.full((NUM, NP_PAD), -1.0, F32), zeros, zeros, zeros, zeros)
    bov, bx1, by1, bx2, by2 = jax.lax.fori_loop(0, NT, body, init, unroll=16)

    pos = bov >= 0.5
    num_pos = jnp.sum(pos.astype(I32), axis=1, keepdims=True)
    n_total = jnp.sum(num_pos).astype(F32)

    lx = loc_ref[0]
    ly = loc_ref[1]
    lw = loc_ref[2]
    lh = loc_ref[3]
    bcx = cx + lx * V0 * w
    bcy = cy + ly * V0 * h
    bw = w * jnp.exp(lw * V1)
    bh = h * jnp.exp(lh * V1)
    dx1 = bcx - bw * 0.5
    dy1 = bcy - bh * 0.5
    dx2 = dx1 + bw
    dy2 = dy1 + bh
    a1 = (dx2 - dx1) * (dy2 - dy1)
    a2 = (bx2 - bx1) * (by2 - by1)
    iw2 = jnp.maximum(jnp.minimum(dx2, bx2) - jnp.maximum(dx1, bx1), 0.0)
    ih2 = jnp.maximum(jnp.minimum(dy2, by2) - jnp.maximum(dy1, by1), 0.0)
    inter2 = iw2 * ih2
    union2 = a1 + a2 - inter2
    iou = inter2 / jnp.maximum(union2, 1e-10)
    ew = jnp.maximum(jnp.maximum(dx2, bx2) - jnp.minimum(dx1, bx1), 0.0)
    eh = jnp.maximum(jnp.maximum(dy2, by2) - jnp.minimum(dy1, by1), 0.0)
    enc = ew * eh
    giou = iou - (enc - union2) / jnp.maximum(enc, 1e-10)
    loss_l_sum = jnp.sum(jnp.where(pos, 1.0 - giou, 0.0))

    x0 = conf_ref[0]
    x1 = conf_ref[1]
    mx = jnp.maximum(x0, x1)
    lse = mx + jnp.log(1.0 + jnp.exp(-jnp.abs(x0 - x1)))
    v = jnp.where(pos | jnp.logical_not(valid), 0.0, lse - x0)
    nll_pos_sum = jnp.sum(jnp.where(pos, lse - x1, 0.0))

    k = jnp.minimum(3 * num_pos, NP_REAL - 1)
    vb = jax.lax.bitcast_convert_type(v, I32)
    r = jnp.zeros((NUM, 1), I32)
    for bit in range(30, -1, -1):
        c = r | jnp.int32(1 << bit)
        cnt = jnp.sum((vb >= c).astype(I32), axis=1, keepdims=True)
        r = jnp.where(cnt >= k, c, r)
    tf = jax.lax.bitcast_convert_type(r, F32)
    gt = v > tf
    sum_gt = jnp.sum(jnp.where(gt, v, 0.0), axis=1, keepdims=True)
    cnt_gt = jnp.sum(gt.astype(F32), axis=1, keepdims=True)
    stop = jnp.where(k > 0, sum_gt + (k.astype(F32) - cnt_gt) * tf, 0.0)
    topk_total = jnp.sum(stop)

    df = f1_ref[...] - f2_ref[...]
    s = jnp.sum(df * df, axis=1, keepdims=True)
    d = jnp.sqrt(s + 1e-9)
    pos_l = d * d
    neg_l = jnp.maximum(1.0 - d, 0.0)
    neg_l = neg_l * neg_l
    w_pos = 1.0 - jnp.exp(-pos_l)
    w_pos = w_pos * w_pos
    w_neg = 1.0 - jnp.exp(-neg_l)
    w_neg = w_neg * w_neg
    labv = lab_ref[NUM - 1:NUM, 0:1]
    fdc = jnp.sum(labv * w_pos * pos_l +
                  (1.0 - labv) * w_neg * neg_l) / jnp.float32(NUM)

    loss_l_o = 5.0 * loss_l_sum / n_total
    loss_c_o = (nll_pos_sum + topk_total + fdc) / n_total
    fdc_o = fdc / n_total

    lanev = jax.lax.broadcasted_iota(I32, (8, 128), 1)
    outv = jnp.where(lanev == 0, loss_l_o,
                     jnp.where(lanev == 1, loss_c_o,
                               jnp.where(lanev == 2, fdc_o, 0.0)))
    out_ref[...] = outv


def kernel(loc_data, conf_data, priors, f_img, f_img_origin, targets):
    pad = NP_PAD - NP_REAL
    padcol = jnp.array([[-10.0], [-10.0], [0.05], [0.05]], F32)
    pr = jnp.concatenate(
        [priors.T.astype(F32), jnp.broadcast_to(padcol, (4, pad))], axis=1)
    loc_t = jnp.pad(jnp.moveaxis(loc_data, 2, 0), ((0, 0), (0, 0), (0, pad)))
    conf_t = jnp.pad(jnp.moveaxis(conf_data, 2, 0), ((0, 0), (0, 0), (0, pad)))
    tr = jnp.transpose(targets[..., :4], (1, 0, 2))
    lab = targets[..., 4]
    out = pl.pallas_call(
        _mb_kernel,
        out_shape=jax.ShapeDtypeStruct((8, 128), F32),
    )(pr, tr, loc_t, conf_t, f_img, f_img_origin, lab)
    return (out[0, 0], out[0, 1], out[0, 2])


# speedup vs baseline: 1.0207x; 1.0207x over previous
import jax
import jax.numpy as jnp
from jax.experimental import pallas as pl

NP_REAL = 8732
NP_PAD = 8832
NUM = 32
NT = 64
V0 = 0.1
V1 = 0.2
F32 = jnp.float32
I32 = jnp.int32


def _mb_kernel(pr_ref, tr_ref, loc_ref, conf_ref, f1_ref, f2_ref, lab_ref,
               out_ref):
    cx = pr_ref[0:1, :]
    cy = pr_ref[1:2, :]
    w = pr_ref[2:3, :]
    h = pr_ref[3:4, :]
    px1 = cx - w * 0.5
    py1 = cy - h * 0.5
    px2 = cx + w * 0.5
    py2 = cy + h * 0.5
    area_b = (px2 - px1) * (py2 - py1)

    def body(t, carry):
        bov, bx1, by1, bx2, by2 = carry
        tb = tr_ref[t]
        ax1 = tb[:, 0:1]
        ay1 = tb[:, 1:2]
        ax2 = tb[:, 2:3]
        ay2 = tb[:, 3:4]
        iw = jnp.maximum(jnp.minimum(ax2, px2) - jnp.maximum(ax1, px1), 0.0)
        ih = jnp.maximum(jnp.minimum(ay2, py2) - jnp.maximum(ay1, py1), 0.0)
        inter = iw * ih
        area_a = (ax2 - ax1) * (ay2 - ay1)
        ov = inter / (area_a + area_b - inter)
        m = jnp.max(ov, axis=1, keepdims=True)
        fm = ov == m
        ovf = jnp.where(fm, 2.0 + t.astype(F32) * 0.0009765625, ov)
        upd = ovf > bov
        bov = jnp.where(upd, ovf, bov)
        bx1 = jnp.where(upd, ax1, bx1)
        by1 = jnp.where(upd, ay1, by1)
        bx2 = jnp.where(upd, ax2, bx2)
        by2 = jnp.where(upd, ay2, by2)
        return bov, bx1, by1, bx2, by2

    zeros = jnp.zeros((NUM, NP_PAD), F32)
    init = (jnp.full((NUM, NP_PAD), -1.0, F32), zeros, zeros, zeros, zeros)
    bov, bx1, by1, bx2, by2 = jax.lax.fori_loop(0, NT, body, init, unroll=8)

    pos = bov >= 0.5
    num_pos = jnp.sum(pos.astype(I32), axis=1, keepdims=True)
    n_total = jnp.sum(num_pos).astype(F32)

    lx = loc_ref[0]
    ly = loc_ref[1]
    lw = loc_ref[2]
    lh = loc_ref[3]
    bcx = cx + lx * V0 * w
    bcy = cy + ly * V0 * h
    bw = w * jnp.exp(lw * V1)
    bh = h * jnp.exp(lh * V1)
    dx1 = bcx - bw * 0.5
    dy1 = bcy - bh * 0.5
    dx2 = dx1 + bw
    dy2 = dy1 + bh
    a1 = (dx2 - dx1) * (dy2 - dy1)
    a2 = (bx2 - bx1) * (by2 - by1)
    iw2 = jnp.maximum(jnp.minimum(dx2, bx2) - jnp.maximum(dx1, bx1), 0.0)
    ih2 = jnp.maximum(jnp.minimum(dy2, by2) - jnp.maximum(dy1, by1), 0.0)
    inter2 = iw2 * ih2
    union2 = a1 + a2 - inter2
    iou = inter2 / jnp.maximum(union2, 1e-10)
    ew = jnp.maximum(jnp.maximum(dx2, bx2) - jnp.minimum(dx1, bx1), 0.0)
    eh = jnp.maximum(jnp.maximum(dy2, by2) - jnp.minimum(dy1, by1), 0.0)
    enc = ew * eh
    giou = iou - (enc - union2) / jnp.maximum(enc, 1e-10)
    loss_l_sum = jnp.sum(jnp.where(pos, 1.0 - giou, 0.0))

    x0 = conf_ref[0]
    x1 = conf_ref[1]
    mx = jnp.maximum(x0, x1)
    lse = mx + jnp.log(1.0 + jnp.exp(-jnp.abs(x0 - x1)))
    v = jnp.where(pos, 0.0, lse - x0)
    nll_pos_sum = jnp.sum(jnp.where(pos, lse - x1, 0.0))

    k = jnp.minimum(3 * num_pos, NP_REAL - 1)
    vb = jax.lax.bitcast_convert_type(v, I32)
    r = jnp.zeros((NUM, 1), I32)
    for bit in range(30, -1, -1):
        c = r | jnp.int32(1 << bit)
        cnt = jnp.sum((vb >= c).astype(I32), axis=1, keepdims=True)
        r = jnp.where(cnt >= k, c, r)
    tf = jax.lax.bitcast_convert_type(r, F32)
    gt = v > tf
    sum_gt = jnp.sum(jnp.where(gt, v, 0.0), axis=1, keepdims=True)
    cnt_gt = jnp.sum(gt.astype(F32), axis=1, keepdims=True)
    stop = jnp.where(k > 0, sum_gt + (k.astype(F32) - cnt_gt) * tf, 0.0)
    topk_total = jnp.sum(stop)

    df = f1_ref[...] - f2_ref[...]
    s = jnp.sum(df * df, axis=1, keepdims=True)
    d = jnp.sqrt(s + 1e-9)
    pos_l = d * d
    neg_l = jnp.maximum(1.0 - d, 0.0)
    neg_l = neg_l * neg_l
    w_pos = 1.0 - jnp.exp(-pos_l)
    w_pos = w_pos * w_pos
    w_neg = 1.0 - jnp.exp(-neg_l)
    w_neg = w_neg * w_neg
    labv = lab_ref[NUM - 1:NUM, 0:1]
    fdc = jnp.sum(labv * w_pos * pos_l +
                  (1.0 - labv) * w_neg * neg_l) / jnp.float32(NUM)

    loss_l_o = 5.0 * loss_l_sum / n_total
    loss_c_o = (nll_pos_sum + topk_total + fdc) / n_total
    fdc_o = fdc / n_total

    lanev = jax.lax.broadcasted_iota(I32, (8, 128), 1)
    outv = jnp.where(lanev == 0, loss_l_o,
                     jnp.where(lanev == 1, loss_c_o,
                               jnp.where(lanev == 2, fdc_o, 0.0)))
    out_ref[...] = outv


def kernel(loc_data, conf_data, priors, f_img, f_img_origin, targets):
    pad = NP_PAD - NP_REAL
    padcol = jnp.array([[-10.0], [-10.0], [0.05], [0.05]], F32)
    pr = jnp.concatenate(
        [priors.T.astype(F32), jnp.broadcast_to(padcol, (4, pad))], axis=1)
    loc_t = jnp.pad(jnp.moveaxis(loc_data, 2, 0), ((0, 0), (0, 0), (0, pad)))
    conf_t = jnp.pad(jnp.moveaxis(conf_data, 2, 0),
                     ((0, 0), (0, 0), (0, pad)))
    conf_t = conf_t.at[1, :, NP_REAL:].set(-1e9)
    tr = jnp.transpose(targets[..., :4], (1, 0, 2))
    lab = targets[..., 4]
    out = pl.pallas_call(
        _mb_kernel,
        out_shape=jax.ShapeDtypeStruct((8, 128), F32),
    )(pr, tr, loc_t, conf_t, f_img, f_img_origin, lab)
    return (out[0, 0], out[0, 1], out[0, 2])


# speedup vs baseline: 1.0290x; 1.0081x over previous
import jax
import jax.numpy as jnp
from jax.experimental import pallas as pl

NP_REAL = 8732
NP_PAD = 8832
NUM = 32
NT = 64
V0 = 0.1
V1 = 0.2
F32 = jnp.float32
I32 = jnp.int32


def _mb_kernel(pr_ref, tr_ref, loc_ref, conf_ref, f1_ref, f2_ref, lab_ref,
               out_ref):
    cx = pr_ref[0:1, :]
    cy = pr_ref[1:2, :]
    w = pr_ref[2:3, :]
    h = pr_ref[3:4, :]
    px1 = cx - w * 0.5
    py1 = cy - h * 0.5
    px2 = cx + w * 0.5
    py2 = cy + h * 0.5
    area_b = (px2 - px1) * (py2 - py1)

    def body(t, carry):
        bov, bx1, by1, bx2, by2 = carry
        tb = tr_ref[t]
        ax1 = tb[:, 0:1]
        ay1 = tb[:, 1:2]
        ax2 = tb[:, 2:3]
        ay2 = tb[:, 3:4]
        iw = jnp.maximum(jnp.minimum(ax2, px2) - jnp.maximum(ax1, px1), 0.0)
        ih = jnp.maximum(jnp.minimum(ay2, py2) - jnp.maximum(ay1, py1), 0.0)
        inter = iw * ih
        area_a = (ax2 - ax1) * (ay2 - ay1)
        ov = inter / (area_a + area_b - inter)
        m = jnp.max(ov, axis=1, keepdims=True)
        fm = ov == m
        ovf = jnp.where(fm, 2.0 + t.astype(F32) * 0.0009765625, ov)
        upd = ovf > bov
        bov = jnp.where(upd, ovf, bov)
        bx1 = jnp.where(upd, ax1, bx1)
        by1 = jnp.where(upd, ay1, by1)
        bx2 = jnp.where(upd, ax2, bx2)
        by2 = jnp.where(upd, ay2, by2)
        return bov, bx1, by1, bx2, by2

    zeros = jnp.zeros((NUM, NP_PAD), F32)
    init = (jnp.full((NUM, NP_PAD), -1.0, F32), zeros, zeros, zeros, zeros)
    bov, bx1, by1, bx2, by2 = jax.lax.fori_loop(0, NT, body, init, unroll=12)

    pos = bov >= 0.5
    num_pos = jnp.sum(pos.astype(I32), axis=1, keepdims=True)
    n_total = jnp.sum(num_pos).astype(F32)

    lx = loc_ref[0]
    ly = loc_ref[1]
    lw = loc_ref[2]
    lh = loc_ref[3]
    bcx = cx + lx * V0 * w
    bcy = cy + ly * V0 * h
    bw = w * jnp.exp(lw * V1)
    bh = h * jnp.exp(lh * V1)
    dx1 = bcx - bw * 0.5
    dy1 = bcy - bh * 0.5
    dx2 = dx1 + bw
    dy2 = dy1 + bh
    a1 = (dx2 - dx1) * (dy2 - dy1)
    a2 = (bx2 - bx1) * (by2 - by1)
    iw2 = jnp.maximum(jnp.minimum(dx2, bx2) - jnp.maximum(dx1, bx1), 0.0)
    ih2 = jnp.maximum(jnp.minimum(dy2, by2) - jnp.maximum(dy1, by1), 0.0)
    inter2 = iw2 * ih2
    union2 = a1 + a2 - inter2
    iou = inter2 / jnp.maximum(union2, 1e-10)
    ew = jnp.maximum(jnp.maximum(dx2, bx2) - jnp.minimum(dx1, bx1), 0.0)
    eh = jnp.maximum(jnp.maximum(dy2, by2) - jnp.minimum(dy1, by1), 0.0)
    enc = ew * eh
    giou = iou - (enc - union2) / jnp.maximum(enc, 1e-10)
    loss_l_sum = jnp.sum(jnp.where(pos, 1.0 - giou, 0.0))

    x0 = conf_ref[0]
    x1 = conf_ref[1]
    mx = jnp.maximum(x0, x1)
    lse = mx + jnp.log(1.0 + jnp.exp(-jnp.abs(x0 - x1)))
    v = jnp.where(pos, 0.0, lse - x0)
    nll_pos_sum = jnp.sum(jnp.where(pos, lse - x1, 0.0))

    k = jnp.minimum(3 * num_pos, NP_REAL - 1)
    vb = jax.lax.bitcast_convert_type(v, I32)
    r = jnp.zeros((NUM, 1), I32)
    for bit in range(30, -1, -1):
        c = r | jnp.int32(1 << bit)
        cnt = jnp.sum((vb >= c).astype(I32), axis=1, keepdims=True)
        r = jnp.where(cnt >= k, c, r)
    tf = jax.lax.bitcast_convert_type(r, F32)
    gt = v > tf
    sum_gt = jnp.sum(jnp.where(gt, v, 0.0), axis=1, keepdims=True)
    cnt_gt = jnp.sum(gt.astype(F32), axis=1, keepdims=True)
    stop = jnp.where(k > 0, sum_gt + (k.astype(F32) - cnt_gt) * tf, 0.0)
    topk_total = jnp.sum(stop)

    df = f1_ref[...] - f2_ref[...]
    s = jnp.sum(df * df, axis=1, keepdims=True)
    d = jnp.sqrt(s + 1e-9)
    pos_l = d * d
    neg_l = jnp.maximum(1.0 - d, 0.0)
    neg_l = neg_l * neg_l
    w_pos = 1.0 - jnp.exp(-pos_l)
    w_pos = w_pos * w_pos
    w_neg = 1.0 - jnp.exp(-neg_l)
    w_neg = w_neg * w_neg
    labv = lab_ref[NUM - 1:NUM, 0:1]
    fdc = jnp.sum(labv * w_pos * pos_l +
                  (1.0 - labv) * w_neg * neg_l) / jnp.float32(NUM)

    loss_l_o = 5.0 * loss_l_sum / n_total
    loss_c_o = (nll_pos_sum + topk_total + fdc) / n_total
    fdc_o = fdc / n_total

    lanev = jax.lax.broadcasted_iota(I32, (8, 128), 1)
    outv = jnp.where(lanev == 0, loss_l_o,
                     jnp.where(lanev == 1, loss_c_o,
                               jnp.where(lanev == 2, fdc_o, 0.0)))
    out_ref[...] = outv


def kernel(loc_data, conf_data, priors, f_img, f_img_origin, targets):
    pad = NP_PAD - NP_REAL
    padcol = jnp.array([[-10.0], [-10.0], [0.05], [0.05]], F32)
    pr = jnp.concatenate(
        [priors.T.astype(F32), jnp.broadcast_to(padcol, (4, pad))], axis=1)
    loc_t = jnp.pad(jnp.moveaxis(loc_data, 2, 0), ((0, 0), (0, 0), (0, pad)))
    conf_t = jnp.pad(jnp.moveaxis(conf_data, 2, 0),
                     ((0, 0), (0, 0), (0, pad)))
    conf_t = conf_t.at[1, :, NP_REAL:].set(-1e9)
    tr = jnp.transpose(targets[..., :4], (1, 0, 2))
    lab = targets[..., 4]
    out = pl.pallas_call(
        _mb_kernel,
        out_shape=jax.ShapeDtypeStruct((8, 128), F32),
    )(pr, tr, loc_t, conf_t, f_img, f_img_origin, lab)
    return (out[0, 0], out[0, 1], out[0, 2])
